# initial kernel scaffold (unmeasured)
import jax
import jax.numpy as jnp
from jax import lax
from jax.experimental import pallas as pl
from jax.experimental.pallas import tpu as pltpu


def kernel(A, B):
    m, k_local = A.shape
    k_local2, n_local = B.shape
    assert k_local == k_local2

    A = A.astype(jnp.bfloat16)
    B = B.astype(jnp.bfloat16)

    def body(a_ref, b_ref, out_ref, recv_ref, send_sem, recv_sem):
        my_x = lax.axis_index("x")
        my_y = lax.axis_index("y")
        peer_x = 1 - my_x

        out_ref[...] = jnp.dot(
            a_ref[...], b_ref[...], preferred_element_type=jnp.float32
        ).astype(jnp.bfloat16)

        rdma = pltpu.make_async_remote_copy(
            src_ref=out_ref,
            dst_ref=recv_ref,
            send_sem=send_sem,
            recv_sem=recv_sem,
            device_id=(peer_x, my_y),
            device_id_type=pl.DeviceIdType.MESH,
        )
        rdma.start()
        rdma.wait()

        out_ref[...] = out_ref[...] + recv_ref[...]

    return pl.pallas_call(
        body,
        out_shape=jax.ShapeDtypeStruct((m, n_local), jnp.bfloat16),
        in_specs=[
            pl.BlockSpec(memory_space=pltpu.VMEM),
            pl.BlockSpec(memory_space=pltpu.VMEM),
        ],
        out_specs=pl.BlockSpec(memory_space=pltpu.VMEM),
        scratch_shapes=[
            pltpu.VMEM((m, n_local), jnp.bfloat16),
            pltpu.SemaphoreType.DMA,
            pltpu.SemaphoreType.DMA,
        ],
    )(A, B)


# baseline (device time: 297228 ns/iter reference)
import jax
import jax.numpy as jnp
from jax import lax
from jax.experimental import pallas as pl
from jax.experimental.pallas import tpu as pltpu

N_CHUNKS = 8


def kernel(A, B):
    m, k_local = A.shape
    k_local2, n_local = B.shape
    assert k_local == k_local2
    assert n_local % N_CHUNKS == 0
    bn = n_local // N_CHUNKS

    A = A.astype(jnp.bfloat16)
    B = B.astype(jnp.bfloat16)

    def body(a_ref, b_ref, out_ref, acc_ref, recv_ref, send_sems, recv_sems):
        my_x = lax.axis_index("x")
        my_y = lax.axis_index("y")
        j = pl.program_id(0)
        slot = lax.rem(j, 2)

        acc_ref[...] = jnp.dot(
            a_ref[...], b_ref[...], preferred_element_type=jnp.float32
        ).astype(jnp.bfloat16)

        rdma = pltpu.make_async_remote_copy(
            src_ref=acc_ref,
            dst_ref=recv_ref.at[slot],
            send_sem=send_sems.at[slot],
            recv_sem=recv_sems.at[slot],
            device_id=(1 - my_x, my_y),
            device_id_type=pl.DeviceIdType.MESH,
        )
        rdma.start()
        rdma.wait()

        out_ref[...] = acc_ref[...] + recv_ref[slot]

    return pl.pallas_call(
        body,
        grid=(N_CHUNKS,),
        out_shape=jax.ShapeDtypeStruct((m, n_local), jnp.bfloat16),
        in_specs=[
            pl.BlockSpec((m, k_local), lambda j: (0, 0), memory_space=pltpu.VMEM),
            pl.BlockSpec((k_local, bn), lambda j: (0, j), memory_space=pltpu.VMEM),
        ],
        out_specs=pl.BlockSpec((m, bn), lambda j: (0, j), memory_space=pltpu.VMEM),
        scratch_shapes=[
            pltpu.VMEM((m, bn), jnp.bfloat16),
            pltpu.VMEM((2, m, bn), jnp.bfloat16),
            pltpu.SemaphoreType.DMA((2,)),
            pltpu.SemaphoreType.DMA((2,)),
        ],
        compiler_params=pltpu.CompilerParams(
            dimension_semantics=("arbitrary",),
        ),
    )(A, B)


# device time: 257745 ns/iter; 1.1532x vs baseline; 1.1532x over previous
import jax
import jax.numpy as jnp
from jax import lax
from jax.experimental import pallas as pl
from jax.experimental.pallas import tpu as pltpu

N_CHUNKS = 8


def kernel(A, B):
    m, k_local = A.shape
    k_local2, n_local = B.shape
    assert k_local == k_local2
    assert n_local % N_CHUNKS == 0
    bn = n_local // N_CHUNKS

    A = A.astype(jnp.bfloat16)
    B = B.astype(jnp.bfloat16)

    def body(a_ref, b_ref, out_ref, acc_ref, recv_ref, send_sems, recv_sems):
        my_x = lax.axis_index("x")
        my_y = lax.axis_index("y")
        j = pl.program_id(0)
        slot = lax.rem(j, 2)
        prev = 1 - slot

        def peer_rdma(s):
            return pltpu.make_async_remote_copy(
                src_ref=acc_ref.at[s],
                dst_ref=recv_ref.at[s],
                send_sem=send_sems.at[s],
                recv_sem=recv_sems.at[s],
                device_id=(1 - my_x, my_y),
                device_id_type=pl.DeviceIdType.MESH,
            )

        @pl.when(j < N_CHUNKS)
        def _():
            acc_ref[slot] = jnp.dot(
                a_ref[...], b_ref[...], preferred_element_type=jnp.float32
            ).astype(jnp.bfloat16)

        @pl.when(j > 0)
        def _():
            peer_rdma(prev).wait()
            out_ref[...] = acc_ref[prev] + recv_ref[prev]

        @pl.when(j < N_CHUNKS)
        def _():
            peer_rdma(slot).start()

    return pl.pallas_call(
        body,
        grid=(N_CHUNKS + 1,),
        out_shape=jax.ShapeDtypeStruct((m, n_local), jnp.bfloat16),
        in_specs=[
            pl.BlockSpec((m, k_local), lambda j: (0, 0), memory_space=pltpu.VMEM),
            pl.BlockSpec(
                (k_local, bn),
                lambda j: (0, jnp.minimum(j, N_CHUNKS - 1)),
                memory_space=pltpu.VMEM,
            ),
        ],
        out_specs=pl.BlockSpec(
            (m, bn), lambda j: (0, jnp.maximum(j - 1, 0)), memory_space=pltpu.VMEM
        ),
        scratch_shapes=[
            pltpu.VMEM((2, m, bn), jnp.bfloat16),
            pltpu.VMEM((2, m, bn), jnp.bfloat16),
            pltpu.SemaphoreType.DMA((2,)),
            pltpu.SemaphoreType.DMA((2,)),
        ],
        compiler_params=pltpu.CompilerParams(
            dimension_semantics=("arbitrary",),
        ),
    )(A, B)


# device time: 244145 ns/iter; 1.2174x vs baseline; 1.0557x over previous
import jax
import jax.numpy as jnp
from jax import lax
from jax.experimental import pallas as pl
from jax.experimental.pallas import tpu as pltpu

N_CHUNKS = 8


def kernel(A, B):
    m, k_local = A.shape
    k_local2, n_local = B.shape
    assert k_local == k_local2
    assert n_local % N_CHUNKS == 0
    bn = n_local // N_CHUNKS

    A = A.astype(jnp.bfloat16)
    B = B.astype(jnp.bfloat16)

    def body(
        a_ref, b_ref, out_ref, acc_ref, recv_ref, send_sems, recv_sems, credit_sems
    ):
        my_x = lax.axis_index("x")
        my_y = lax.axis_index("y")
        peer = (1 - my_x, my_y)
        j = pl.program_id(0)
        slot = lax.rem(j, 2)
        prev = 1 - slot

        def peer_rdma(s):
            return pltpu.make_async_remote_copy(
                src_ref=acc_ref.at[s],
                dst_ref=recv_ref.at[s],
                send_sem=send_sems.at[s],
                recv_sem=recv_sems.at[s],
                device_id=peer,
                device_id_type=pl.DeviceIdType.MESH,
            )

        @pl.when(j < N_CHUNKS)
        def _():
            acc_ref[slot] = jnp.dot(
                a_ref[...], b_ref[...], preferred_element_type=jnp.float32
            ).astype(jnp.bfloat16)

        @pl.when(jnp.logical_and(j >= 2, j < N_CHUNKS))
        def _():
            pl.semaphore_wait(credit_sems.at[slot], 1)

        @pl.when(j < N_CHUNKS)
        def _():
            peer_rdma(slot).start()

        @pl.when(j > 0)
        def _():
            peer_rdma(prev).wait()
            out_ref[...] = acc_ref[prev] + recv_ref[prev]

        @pl.when(jnp.logical_and(j > 0, j <= N_CHUNKS - 2))
        def _():
            pl.semaphore_signal(
                credit_sems.at[prev],
                inc=1,
                device_id=peer,
                device_id_type=pl.DeviceIdType.MESH,
            )

    return pl.pallas_call(
        body,
        grid=(N_CHUNKS + 1,),
        out_shape=jax.ShapeDtypeStruct((m, n_local), jnp.bfloat16),
        in_specs=[
            pl.BlockSpec((m, k_local), lambda j: (0, 0), memory_space=pltpu.VMEM),
            pl.BlockSpec(
                (k_local, bn),
                lambda j: (0, jnp.minimum(j, N_CHUNKS - 1)),
                memory_space=pltpu.VMEM,
            ),
        ],
        out_specs=pl.BlockSpec(
            (m, bn), lambda j: (0, jnp.maximum(j - 1, 0)), memory_space=pltpu.VMEM
        ),
        scratch_shapes=[
            pltpu.VMEM((2, m, bn), jnp.bfloat16),
            pltpu.VMEM((2, m, bn), jnp.bfloat16),
            pltpu.SemaphoreType.DMA((2,)),
            pltpu.SemaphoreType.DMA((2,)),
            pltpu.SemaphoreType.REGULAR((2,)),
        ],
        compiler_params=pltpu.CompilerParams(
            dimension_semantics=("arbitrary",),
        ),
    )(A, B)


# device time: 241258 ns/iter; 1.2320x vs baseline; 1.0120x over previous
import jax
import jax.numpy as jnp
from jax import lax
from jax.experimental import pallas as pl
from jax.experimental.pallas import tpu as pltpu

N_CHUNKS = 12


def kernel(A, B):
    m, k_local = A.shape
    k_local2, n_local = B.shape
    assert k_local == k_local2
    assert n_local % N_CHUNKS == 0
    bn = n_local // N_CHUNKS

    A = A.astype(jnp.bfloat16)
    B = B.astype(jnp.bfloat16)

    def body(
        a_ref, b_ref, out_ref, acc_ref, recv_ref, send_sems, recv_sems, credit_sems
    ):
        my_x = lax.axis_index("x")
        my_y = lax.axis_index("y")
        peer = (1 - my_x, my_y)
        j = pl.program_id(0)
        slot = lax.rem(j, 2)
        prev = 1 - slot

        def peer_rdma(s):
            return pltpu.make_async_remote_copy(
                src_ref=acc_ref.at[s],
                dst_ref=recv_ref.at[s],
                send_sem=send_sems.at[s],
                recv_sem=recv_sems.at[s],
                device_id=peer,
                device_id_type=pl.DeviceIdType.MESH,
            )

        @pl.when(j < N_CHUNKS)
        def _():
            acc_ref[slot] = jnp.dot(
                a_ref[...], b_ref[...], preferred_element_type=jnp.float32
            ).astype(jnp.bfloat16)

        @pl.when(jnp.logical_and(j >= 2, j < N_CHUNKS))
        def _():
            pl.semaphore_wait(credit_sems.at[slot], 1)

        @pl.when(j < N_CHUNKS)
        def _():
            peer_rdma(slot).start()

        @pl.when(j > 0)
        def _():
            peer_rdma(prev).wait()
            out_ref[...] = acc_ref[prev] + recv_ref[prev]

        @pl.when(jnp.logical_and(j > 0, j <= N_CHUNKS - 2))
        def _():
            pl.semaphore_signal(
                credit_sems.at[prev],
                inc=1,
                device_id=peer,
                device_id_type=pl.DeviceIdType.MESH,
            )

    return pl.pallas_call(
        body,
        grid=(N_CHUNKS + 1,),
        out_shape=jax.ShapeDtypeStruct((m, n_local), jnp.bfloat16),
        in_specs=[
            pl.BlockSpec((m, k_local), lambda j: (0, 0), memory_space=pltpu.VMEM),
            pl.BlockSpec(
                (k_local, bn),
                lambda j: (0, jnp.minimum(j, N_CHUNKS - 1)),
                memory_space=pltpu.VMEM,
            ),
        ],
        out_specs=pl.BlockSpec(
            (m, bn), lambda j: (0, jnp.maximum(j - 1, 0)), memory_space=pltpu.VMEM
        ),
        scratch_shapes=[
            pltpu.VMEM((2, m, bn), jnp.bfloat16),
            pltpu.VMEM((2, m, bn), jnp.bfloat16),
            pltpu.SemaphoreType.DMA((2,)),
            pltpu.SemaphoreType.DMA((2,)),
            pltpu.SemaphoreType.REGULAR((2,)),
        ],
        compiler_params=pltpu.CompilerParams(
            dimension_semantics=("arbitrary",),
        ),
    )(A, B)


# device time: 235291 ns/iter; 1.2632x vs baseline; 1.0254x over previous
import jax
import jax.numpy as jnp
from jax import lax
from jax.experimental import pallas as pl
from jax.experimental.pallas import tpu as pltpu

N_CHUNKS = 12


def kernel(A, B):
    m, k_local = A.shape
    k_local2, n_local = B.shape
    assert k_local == k_local2
    assert n_local % N_CHUNKS == 0
    bn = n_local // N_CHUNKS

    A = A.astype(jnp.bfloat16)

    def body(
        a_ref, b_ref, out_ref, acc_ref, recv_ref, send_sems, recv_sems, credit_sems
    ):
        my_x = lax.axis_index("x")
        my_y = lax.axis_index("y")
        peer = (1 - my_x, my_y)
        j = pl.program_id(0)
        slot = lax.rem(j, 2)
        prev = 1 - slot

        def peer_rdma(s):
            return pltpu.make_async_remote_copy(
                src_ref=acc_ref.at[s],
                dst_ref=recv_ref.at[s],
                send_sem=send_sems.at[s],
                recv_sem=recv_sems.at[s],
                device_id=peer,
                device_id_type=pl.DeviceIdType.MESH,
            )

        @pl.when(j < N_CHUNKS)
        def _():
            acc_ref[slot] = jnp.dot(
                a_ref[...],
                b_ref[...].astype(jnp.bfloat16),
                preferred_element_type=jnp.float32,
            ).astype(jnp.bfloat16)

        @pl.when(jnp.logical_and(j >= 2, j < N_CHUNKS))
        def _():
            pl.semaphore_wait(credit_sems.at[slot], 1)

        @pl.when(j < N_CHUNKS)
        def _():
            peer_rdma(slot).start()

        @pl.when(j > 0)
        def _():
            peer_rdma(prev).wait()
            out_ref[...] = acc_ref[prev] + recv_ref[prev]

        @pl.when(jnp.logical_and(j > 0, j <= N_CHUNKS - 2))
        def _():
            pl.semaphore_signal(
                credit_sems.at[prev],
                inc=1,
                device_id=peer,
                device_id_type=pl.DeviceIdType.MESH,
            )

    return pl.pallas_call(
        body,
        grid=(N_CHUNKS + 1,),
        out_shape=jax.ShapeDtypeStruct((m, n_local), jnp.bfloat16),
        in_specs=[
            pl.BlockSpec((m, k_local), lambda j: (0, 0), memory_space=pltpu.VMEM),
            pl.BlockSpec(
                (k_local, bn),
                lambda j: (0, jnp.minimum(j, N_CHUNKS - 1)),
                memory_space=pltpu.VMEM,
            ),
        ],
        out_specs=pl.BlockSpec(
            (m, bn), lambda j: (0, jnp.maximum(j - 1, 0)), memory_space=pltpu.VMEM
        ),
        scratch_shapes=[
            pltpu.VMEM((2, m, bn), jnp.bfloat16),
            pltpu.VMEM((2, m, bn), jnp.bfloat16),
            pltpu.SemaphoreType.DMA((2,)),
            pltpu.SemaphoreType.DMA((2,)),
            pltpu.SemaphoreType.REGULAR((2,)),
        ],
        compiler_params=pltpu.CompilerParams(
            dimension_semantics=("arbitrary",),
        ),
    )(A, B)


# device time: 234407 ns/iter; 1.2680x vs baseline; 1.0038x over previous
import jax
import jax.numpy as jnp
from jax import lax
from jax.experimental import pallas as pl
from jax.experimental.pallas import tpu as pltpu

N_CHUNKS = 24


def kernel(A, B):
    m, k_local = A.shape
    k_local2, n_local = B.shape
    assert k_local == k_local2
    assert n_local % N_CHUNKS == 0
    bn = n_local // N_CHUNKS

    A = A.astype(jnp.bfloat16)

    def body(
        a_ref, b_ref, out_ref, acc_ref, recv_ref, send_sems, recv_sems, credit_sems
    ):
        my_x = lax.axis_index("x")
        my_y = lax.axis_index("y")
        peer = (1 - my_x, my_y)
        j = pl.program_id(0)
        slot = lax.rem(j, 2)
        prev = 1 - slot

        def peer_rdma(s):
            return pltpu.make_async_remote_copy(
                src_ref=acc_ref.at[s],
                dst_ref=recv_ref.at[s],
                send_sem=send_sems.at[s],
                recv_sem=recv_sems.at[s],
                device_id=peer,
                device_id_type=pl.DeviceIdType.MESH,
            )

        @pl.when(j < N_CHUNKS)
        def _():
            acc_ref[slot] = jnp.dot(
                a_ref[...],
                b_ref[...].astype(jnp.bfloat16),
                preferred_element_type=jnp.float32,
            ).astype(jnp.bfloat16)

        @pl.when(jnp.logical_and(j >= 2, j < N_CHUNKS))
        def _():
            pl.semaphore_wait(credit_sems.at[slot], 1)

        @pl.when(j < N_CHUNKS)
        def _():
            peer_rdma(slot).start()

        @pl.when(j > 0)
        def _():
            peer_rdma(prev).wait()
            out_ref[...] = acc_ref[prev] + recv_ref[prev]

        @pl.when(jnp.logical_and(j > 0, j <= N_CHUNKS - 2))
        def _():
            pl.semaphore_signal(
                credit_sems.at[prev],
                inc=1,
                device_id=peer,
                device_id_type=pl.DeviceIdType.MESH,
            )

    return pl.pallas_call(
        body,
        grid=(N_CHUNKS + 1,),
        out_shape=jax.ShapeDtypeStruct((m, n_local), jnp.bfloat16),
        in_specs=[
            pl.BlockSpec((m, k_local), lambda j: (0, 0), memory_space=pltpu.VMEM),
            pl.BlockSpec(
                (k_local, bn),
                lambda j: (0, jnp.minimum(j, N_CHUNKS - 1)),
                memory_space=pltpu.VMEM,
            ),
        ],
        out_specs=pl.BlockSpec(
            (m, bn), lambda j: (0, jnp.maximum(j - 1, 0)), memory_space=pltpu.VMEM
        ),
        scratch_shapes=[
            pltpu.VMEM((2, m, bn), jnp.bfloat16),
            pltpu.VMEM((2, m, bn), jnp.bfloat16),
            pltpu.SemaphoreType.DMA((2,)),
            pltpu.SemaphoreType.DMA((2,)),
            pltpu.SemaphoreType.REGULAR((2,)),
        ],
        compiler_params=pltpu.CompilerParams(
            dimension_semantics=("arbitrary",),
        ),
    )(A, B)
